# Initial kernel scaffold; baseline (speedup 1.0000x reference)
import jax
import jax.numpy as jnp
from jax.experimental import pallas as pl


def kernel(x, edge_index, edge_attr, W1, b1, W2, b2, Wf, bf):
    raise NotImplementedError("write your pallas kernel here")


# speedup vs baseline: 5.0268x; 5.0268x over previous
import functools

import jax
import jax.numpy as jnp
from jax import lax
from jax.experimental import pallas as pl
from jax.experimental.pallas import tpu as pltpu
from jax.experimental.pallas import tpu_sc as plsc

NN = 10000
EE = 320000
DNF = 128
DEF = 4
LL = 6
HH = 16
NCV = 8

NC = 2
NS = 16
NW = NC * NS
NPT = NN // NS
EPW = EE // NW
CK = 80
NCHUNK = EPW // CK

_F32 = jnp.float32


_BF16 = jnp.bfloat16


def _bf(v):
    return v.astype(_BF16).astype(_F32)


def _xp_body(x_ref, w_ref, o_ref):
    o_ref[0] = jnp.dot(x_ref[...].astype(_BF16), w_ref[0].astype(_BF16),
                       preferred_element_type=_F32)


def _xp_call(x, w1x):
    return pl.pallas_call(
        _xp_body,
        grid=(NCV,),
        in_specs=[
            pl.BlockSpec((NN, DNF), lambda i: (0, 0)),
            pl.BlockSpec((1, DNF, HH), lambda i: (i, 0, 0)),
        ],
        out_specs=pl.BlockSpec((1, NN, HH), lambda i: (i, 0, 0)),
        out_shape=jax.ShapeDtypeStruct((NCV, NN, HH), _F32),
    )(x, w1x)


def _ep_body(ea_ref, w_ref, b_ref, o_ref):
    o_ref[0] = (jnp.dot(ea_ref[...].astype(_BF16), w_ref[0].astype(_BF16),
                        preferred_element_type=_F32)
                + b_ref[0])


_EPK = EE // 8
_EBLKP = 5000


def _ep_call(ea_packed, wbig, btile):
    return pl.pallas_call(
        _ep_body,
        grid=(NCV, _EPK // _EBLKP),
        in_specs=[
            pl.BlockSpec((_EBLKP, 8 * DEF), lambda i, e: (e, 0)),
            pl.BlockSpec((1, 8 * DEF, 8 * HH), lambda i, e: (i, 0, 0)),
            pl.BlockSpec((1, 1, 8 * HH), lambda i, e: (i, 0, 0)),
        ],
        out_specs=pl.BlockSpec((1, _EBLKP, 8 * HH), lambda i, e: (i, e, 0)),
        out_shape=jax.ShapeDtypeStruct((NCV, _EPK, 8 * HH), _F32),
    )(ea_packed, wbig, btile)


def _fma_mm(a, w_ref, k, round_a=True):
    av = _bf(a) if round_a else a
    acc = av[:, 0:1] * _bf(w_ref[0:1, :])
    for j in range(1, k):
        acc = acc + av[:, j:j + 1] * _bf(w_ref[j:j + 1, :])
    return acc


def _node_body(hag_ref, cnt_ref, xp_ref, w1a_ref, w1b_ref, w2_ref, b2_ref,
               ab_ref):
    hs = hag_ref[0] + hag_ref[1]
    cnt = jnp.maximum(cnt_ref[0, :, 0:1] + cnt_ref[1, :, 0:1], 1.0)
    xn = jax.nn.relu(_fma_mm(hs / cnt, w2_ref, HH, round_a=False)
                     + b2_ref[0:1, :])
    ab_ref[0] = _fma_mm(xn, w1a_ref, LL) + xp_ref[...]
    ab_ref[1] = _fma_mm(xn, w1b_ref, LL)


_NBLK = 2000


def _node_call(hag, cnt16, xp, w1a, w1b, w2, b2):
    return pl.pallas_call(
        _node_body,
        grid=(NN // _NBLK,),
        in_specs=[
            pl.BlockSpec((2, _NBLK, HH), lambda g: (0, g, 0)),
            pl.BlockSpec((2, _NBLK, HH), lambda g: (0, g, 0)),
            pl.BlockSpec((_NBLK, HH), lambda g: (g, 0)),
            pl.BlockSpec((LL, HH), lambda g: (0, 0)),
            pl.BlockSpec((LL, HH), lambda g: (0, 0)),
            pl.BlockSpec((HH, LL), lambda g: (0, 0)),
            pl.BlockSpec((1, LL), lambda g: (0, 0)),
        ],
        out_specs=pl.BlockSpec((2, _NBLK, HH), lambda g: (0, g, 0)),
        out_shape=jax.ShapeDtypeStruct((2, NN, HH), _F32),
    )(hag, cnt16, xp, w1a, w1b, w2, b2)


def _fin_node_body(hag_ref, cnt_ref, w2_ref, b2_ref, wf_ref, bf_ref, xl_ref,
                   p_ref, pb_ref):
    hs = hag_ref[0] + hag_ref[1]
    cnt = jnp.maximum(cnt_ref[0, :, 0:1] + cnt_ref[1, :, 0:1], 1.0)
    xn = jax.nn.relu(_fma_mm(hs / cnt, w2_ref, HH, round_a=False)
                     + b2_ref[0:1, :])
    p = jax.nn.relu(_fma_mm(xn, wf_ref, LL) + bf_ref[0:1, :])
    p_ref[...] = p
    xl = xl_ref[...]
    pb_ref[...] = jnp.where(xl != 0.0, xl, p)


def _fin_node_call(hag, cnt16, w2, b2, wf, bf, xlast):
    return pl.pallas_call(
        _fin_node_body,
        grid=(NN // _NBLK,),
        in_specs=[
            pl.BlockSpec((2, _NBLK, HH), lambda g: (0, g, 0)),
            pl.BlockSpec((2, _NBLK, HH), lambda g: (0, g, 0)),
            pl.BlockSpec((HH, LL), lambda g: (0, 0)),
            pl.BlockSpec((1, LL), lambda g: (0, 0)),
            pl.BlockSpec((LL, 1), lambda g: (0, 0)),
            pl.BlockSpec((1, 1), lambda g: (0, 0)),
            pl.BlockSpec((_NBLK, 1), lambda g: (g, 0)),
        ],
        out_specs=(
            pl.BlockSpec((_NBLK, 1), lambda g: (g, 0)),
            pl.BlockSpec((_NBLK, 1), lambda g: (g, 0)),
        ),
        out_shape=(
            jax.ShapeDtypeStruct((NN, 1), _F32),
            jax.ShapeDtypeStruct((NN, 1), _F32),
        ),
    )(hag, cnt16, w2, b2, wf, bf, xlast)


def _imb_body(pb_ref, np_ref, o_ref):
    net = pb_ref[...] + np_ref[0, :, 0:1] + np_ref[1, :, 0:1]
    o_ref[...] = jnp.sum(jnp.abs(net), keepdims=True)


def _imb_call(pb, netp):
    return pl.pallas_call(
        _imb_body,
        out_shape=jax.ShapeDtypeStruct((1, 1), _F32),
    )(pb, netp)


def _zero_rows(buf, nrows):
    z = jnp.zeros((HH,), _F32)

    def body(r, _):
        buf[r] = z
        return 0

    lax.fori_loop(0, nrows, body, 0)


def _make_edge_kernel(with_cnt):
    mesh = plsc.VectorSubcoreMesh(core_axis_name="c", subcore_axis_name="s", num_cores=NC, num_subcores=NS)
    out_type = [jax.ShapeDtypeStruct((2, NN, HH), _F32)]
    scratch = [
        pltpu.VMEM_SHARED((NN, HH), _F32),
        pltpu.VMEM_SHARED((NN, HH), _F32),
        pltpu.VMEM_SHARED((NN, HH), _F32),
        pltpu.VMEM((NPT, HH), _F32),
        pltpu.VMEM((CK,), jnp.int32),
        pltpu.VMEM((CK,), jnp.int32),
        pltpu.VMEM((CK, HH), _F32),
        pltpu.VMEM((CK, HH), _F32),
        pltpu.VMEM((CK, HH), _F32),
        pltpu.VMEM((CK, HH), _F32),
        pltpu.SemaphoreType.DMA,
    ]
    if with_cnt:
        out_type.append(jax.ShapeDtypeStruct((2, NN, HH), _F32))
        scratch.append(pltpu.VMEM_SHARED((NN, HH), _F32))
        scratch.append(pltpu.VMEM((CK, HH), _F32))

    def body(ab_hbm, src_hbm, dst_hbm, ep_hbm, *rest):
        if with_cnt:
            (hag_out, cnt_out, at, bt, hg, stage, sidx, didx, epb, ga, gb, hb,
             sem, cg, ones) = rest
        else:
            (hag_out, at, bt, hg, stage, sidx, didx, epb, ga, gb, hb,
             sem) = rest
        c = lax.axis_index("c")
        s = lax.axis_index("s")
        nb = s * NPT
        pltpu.sync_copy(ab_hbm.at[0, pl.ds(nb, NPT)], stage)
        pltpu.sync_copy(stage, at.at[pl.ds(nb, NPT)])
        pltpu.sync_copy(ab_hbm.at[1, pl.ds(nb, NPT)], stage)
        pltpu.sync_copy(stage, bt.at[pl.ds(nb, NPT)])
        _zero_rows(stage, NPT)
        pltpu.sync_copy(stage, hg.at[pl.ds(nb, NPT)])
        if with_cnt:
            pltpu.sync_copy(stage, cg.at[pl.ds(nb, NPT)])
            one = jnp.ones((HH,), _F32)

            def fill_ones(r, _):
                ones[r] = one
                return 0

            lax.fori_loop(0, CK, fill_ones, 0)
        plsc.subcore_barrier()

        eb0 = (c * NS + s) * EPW

        def chunk(ci, _):
            base = eb0 + ci * CK
            pltpu.sync_copy(src_hbm.at[pl.ds(base, CK)], sidx)
            pltpu.sync_copy(dst_hbm.at[pl.ds(base, CK)], didx)
            pltpu.sync_copy(ep_hbm.at[pl.ds(base, CK)], epb)
            pltpu.async_copy(at.at[sidx], ga, sem).wait()
            pltpu.async_copy(bt.at[didx], gb, sem).wait()

            def edge(k, _):
                h = jnp.maximum(ga[k] + gb[k] + epb[k], 0.0)
                u = lax.bitcast_convert_type(h, jnp.int32)
                u = u + 0x7FFF + ((u >> 16) & 1)
                u = u & jnp.int32(-65536)
                hb[k] = lax.bitcast_convert_type(u, _F32)
                return 0

            lax.fori_loop(0, CK, edge, 0)
            pltpu.sync_copy(hb, hg.at[didx], add=True)
            if with_cnt:
                pltpu.sync_copy(ones, cg.at[didx], add=True)
            return 0

        lax.fori_loop(0, NCHUNK, chunk, 0)
        plsc.subcore_barrier()
        pltpu.sync_copy(hg.at[pl.ds(nb, NPT)], stage)
        pltpu.sync_copy(stage, hag_out.at[c, pl.ds(nb, NPT)])
        if with_cnt:
            pltpu.sync_copy(cg.at[pl.ds(nb, NPT)], stage)
            pltpu.sync_copy(stage, cnt_out.at[c, pl.ds(nb, NPT)])

    return pl.kernel(
        body,
        out_type=tuple(out_type) if with_cnt else out_type[0],
        mesh=mesh,
        scratch_types=scratch,
        compiler_params=pltpu.CompilerParams(use_tc_tiling_on_sc=False, needs_layout_passes=False),
    )


def _make_flow_kernel():
    mesh = plsc.VectorSubcoreMesh(core_axis_name="c", subcore_axis_name="s", num_cores=NC, num_subcores=NS)
    scratch = [
        pltpu.VMEM_SHARED((NN, HH), _F32),
        pltpu.VMEM((NN,), _F32),
        pltpu.VMEM((NPT, HH), _F32),
        pltpu.VMEM((CK,), jnp.int32),
        pltpu.VMEM((CK,), jnp.int32),
        pltpu.VMEM((CK,), _F32),
        pltpu.VMEM((CK,), _F32),
        pltpu.VMEM((CK, HH), _F32),
        pltpu.VMEM((CK, HH), _F32),
    ]

    def body(pb_hbm, src_hbm, dst_hbm, ea0_hbm, flow_out, netp_out,
             neta, pbuf, stage, sidx, didx, ea0b, flb, posr, negr):
        c = lax.axis_index("c")
        s = lax.axis_index("s")
        nb = s * NPT
        pltpu.sync_copy(pb_hbm, pbuf)
        _zero_rows(stage, NPT)
        pltpu.sync_copy(stage, neta.at[pl.ds(nb, NPT)])
        plsc.subcore_barrier()

        eb0 = (c * NS + s) * EPW

        def chunk(ci, _):
            base = eb0 + ci * CK
            pltpu.sync_copy(src_hbm.at[pl.ds(base, CK)], sidx)
            pltpu.sync_copy(dst_hbm.at[pl.ds(base, CK)], didx)
            pltpu.sync_copy(ea0_hbm.at[pl.ds(base, CK)], ea0b)
            for j in range(CK // HH):
                sv = sidx[pl.ds(j * HH, HH)]
                dv = didx[pl.ds(j * HH, HH)]
                ps = plsc.load_gather(pbuf, [sv])
                pd = plsc.load_gather(pbuf, [dv])
                fl = (ps - pd) * ea0b[pl.ds(j * HH, HH)]
                flb[pl.ds(j * HH, HH)] = fl
                for e in range(HH):
                    fv = jnp.broadcast_to(fl[e], (HH,))
                    posr[j * HH + e] = fv
                    negr[j * HH + e] = -fv
            pltpu.sync_copy(flb, flow_out.at[pl.ds(base, CK)])
            pltpu.sync_copy(posr, neta.at[didx], add=True)
            pltpu.sync_copy(negr, neta.at[sidx], add=True)
            return 0

        lax.fori_loop(0, NCHUNK, chunk, 0)
        plsc.subcore_barrier()
        pltpu.sync_copy(neta.at[pl.ds(nb, NPT)], stage)
        pltpu.sync_copy(stage, netp_out.at[c, pl.ds(nb, NPT)])

    return pl.kernel(
        body,
        out_type=(
            jax.ShapeDtypeStruct((EE,), _F32),
            jax.ShapeDtypeStruct((2, NN, HH), _F32),
        ),
        mesh=mesh,
        scratch_types=scratch,
        compiler_params=pltpu.CompilerParams(use_tc_tiling_on_sc=False, needs_layout_passes=False),
    )


_SC_KERNELS = {}


def _get_sc_kernels():
    if not _SC_KERNELS:
        _SC_KERNELS["edge_first"] = _make_edge_kernel(with_cnt=True)
        _SC_KERNELS["edge_rest"] = _make_edge_kernel(with_cnt=False)
        _SC_KERNELS["flow"] = _make_flow_kernel()
    return _SC_KERNELS


def kernel(x, edge_index, edge_attr, W1, b1, W2, b2, Wf, bf):
    src = edge_index[0]
    dst = edge_index[1]
    w1a = W1[:, :LL, :]
    w1b = W1[:, LL:2 * LL, :]
    w1x = W1[:, 2 * LL:2 * LL + DNF, :]
    w1e = W1[:, 2 * LL + DNF:, :]
    xlast = x[:, DNF - 1:DNF]
    ea0 = edge_attr[:, 0]

    sck = _get_sc_kernels()
    xp = _xp_call(x, w1x)
    eap = edge_attr.reshape(_EPK, 8 * DEF)
    wbig = jnp.stack(
        [jax.scipy.linalg.block_diag(*([w1e[i]] * 8)) for i in range(NCV)])
    btile = jnp.tile(b1, (1, 8)).reshape(NCV, 1, 8 * HH)
    ep_all = _ep_call(eap, wbig, btile).reshape(NCV, EE, HH)

    ab = jnp.stack([xp[0], jnp.zeros((NN, HH), _F32)])
    hag, cnt16 = sck["edge_first"](ab, src, dst, ep_all[0])
    for i in range(1, NCV):
        ab = _node_call(hag, cnt16, xp[i], w1a[i], w1b[i], W2[i - 1],
                        b2[i - 1].reshape(1, LL))
        hag = sck["edge_rest"](ab, src, dst, ep_all[i])

    p, pb = _fin_node_call(hag, cnt16, W2[NCV - 1], b2[NCV - 1].reshape(1, LL),
                           Wf, bf.reshape(1, 1), xlast)

    flow, netp = sck["flow"](pb.reshape(NN), src, dst, ea0)
    imb = _imb_call(pb, netp)
    return p, flow.reshape(EE, 1), imb.reshape(1)


# speedup vs baseline: 6.3176x; 1.2568x over previous
import functools

import jax
import jax.numpy as jnp
from jax import lax
from jax.experimental import pallas as pl
from jax.experimental.pallas import tpu as pltpu
from jax.experimental.pallas import tpu_sc as plsc

NN = 10000
EE = 320000
DNF = 128
DEF = 4
LL = 6
HH = 16
NCV = 8

NC = 2
NS = 16
NW = NC * NS
NPT = NN // NS
EPW = EE // NW
CK = 80
NCHUNK = EPW // CK

_F32 = jnp.float32


_BF16 = jnp.bfloat16


def _bf(v):
    return v.astype(_BF16).astype(_F32)


def _xp_body(x_ref, w_ref, o_ref):
    o_ref[0] = jnp.dot(x_ref[...].astype(_BF16), w_ref[0].astype(_BF16),
                       preferred_element_type=_F32)


def _xp_call(x, w1x):
    return pl.pallas_call(
        _xp_body,
        grid=(NCV,),
        in_specs=[
            pl.BlockSpec((NN, DNF), lambda i: (0, 0)),
            pl.BlockSpec((1, DNF, HH), lambda i: (i, 0, 0)),
        ],
        out_specs=pl.BlockSpec((1, NN, HH), lambda i: (i, 0, 0)),
        out_shape=jax.ShapeDtypeStruct((NCV, NN, HH), _F32),
    )(x, w1x)


def _ep_body(ea_ref, w_ref, b_ref, o_ref):
    o_ref[0] = (jnp.dot(ea_ref[...].astype(_BF16), w_ref[0].astype(_BF16),
                        preferred_element_type=_F32)
                + b_ref[0])


_EPK = EE // 8
_EBLKP = 5000


def _ep_call(ea_packed, wbig, btile):
    return pl.pallas_call(
        _ep_body,
        grid=(NCV, _EPK // _EBLKP),
        in_specs=[
            pl.BlockSpec((_EBLKP, 8 * DEF), lambda i, e: (e, 0)),
            pl.BlockSpec((1, 8 * DEF, 8 * HH), lambda i, e: (i, 0, 0)),
            pl.BlockSpec((1, 1, 8 * HH), lambda i, e: (i, 0, 0)),
        ],
        out_specs=pl.BlockSpec((1, _EBLKP, 8 * HH), lambda i, e: (i, e, 0)),
        out_shape=jax.ShapeDtypeStruct((NCV, _EPK, 8 * HH), _F32),
    )(ea_packed, wbig, btile)


def _fma_mm(a, w_ref, k, round_a=True):
    av = _bf(a) if round_a else a
    acc = av[:, 0:1] * _bf(w_ref[0:1, :])
    for j in range(1, k):
        acc = acc + av[:, j:j + 1] * _bf(w_ref[j:j + 1, :])
    return acc


def _node_body(hag_ref, cnt_ref, xp_ref, w1a_ref, w1b_ref, w2_ref, b2_ref,
               ab_ref):
    hs = hag_ref[0] + hag_ref[1]
    cnt = jnp.maximum(cnt_ref[0, :, 0:1] + cnt_ref[1, :, 0:1], 1.0)
    xn = jax.nn.relu(_fma_mm(hs / cnt, w2_ref, HH, round_a=False)
                     + b2_ref[0:1, :])
    ab_ref[0] = _fma_mm(xn, w1a_ref, LL) + xp_ref[...]
    ab_ref[1] = _fma_mm(xn, w1b_ref, LL)


_NBLK = 2000


def _node_call(hag, cnt16, xp, w1a, w1b, w2, b2):
    return pl.pallas_call(
        _node_body,
        grid=(NN // _NBLK,),
        in_specs=[
            pl.BlockSpec((2, _NBLK, HH), lambda g: (0, g, 0)),
            pl.BlockSpec((2, _NBLK, HH), lambda g: (0, g, 0)),
            pl.BlockSpec((_NBLK, HH), lambda g: (g, 0)),
            pl.BlockSpec((LL, HH), lambda g: (0, 0)),
            pl.BlockSpec((LL, HH), lambda g: (0, 0)),
            pl.BlockSpec((HH, LL), lambda g: (0, 0)),
            pl.BlockSpec((1, LL), lambda g: (0, 0)),
        ],
        out_specs=pl.BlockSpec((2, _NBLK, HH), lambda g: (0, g, 0)),
        out_shape=jax.ShapeDtypeStruct((2, NN, HH), _F32),
    )(hag, cnt16, xp, w1a, w1b, w2, b2)


def _fin_node_body(hag_ref, cnt_ref, w2_ref, b2_ref, wf_ref, bf_ref, xl_ref,
                   p_ref, pb_ref):
    hs = hag_ref[0] + hag_ref[1]
    cnt = jnp.maximum(cnt_ref[0, :, 0:1] + cnt_ref[1, :, 0:1], 1.0)
    xn = jax.nn.relu(_fma_mm(hs / cnt, w2_ref, HH, round_a=False)
                     + b2_ref[0:1, :])
    p = jax.nn.relu(_fma_mm(xn, wf_ref, LL) + bf_ref[0:1, :])
    p_ref[...] = p
    xl = xl_ref[...]
    pb_ref[...] = jnp.where(xl != 0.0, xl, p)


def _fin_node_call(hag, cnt16, w2, b2, wf, bf, xlast):
    return pl.pallas_call(
        _fin_node_body,
        grid=(NN // _NBLK,),
        in_specs=[
            pl.BlockSpec((2, _NBLK, HH), lambda g: (0, g, 0)),
            pl.BlockSpec((2, _NBLK, HH), lambda g: (0, g, 0)),
            pl.BlockSpec((HH, LL), lambda g: (0, 0)),
            pl.BlockSpec((1, LL), lambda g: (0, 0)),
            pl.BlockSpec((LL, 1), lambda g: (0, 0)),
            pl.BlockSpec((1, 1), lambda g: (0, 0)),
            pl.BlockSpec((_NBLK, 1), lambda g: (g, 0)),
        ],
        out_specs=(
            pl.BlockSpec((_NBLK, 1), lambda g: (g, 0)),
            pl.BlockSpec((_NBLK, 1), lambda g: (g, 0)),
        ),
        out_shape=(
            jax.ShapeDtypeStruct((NN, 1), _F32),
            jax.ShapeDtypeStruct((NN, 1), _F32),
        ),
    )(hag, cnt16, w2, b2, wf, bf, xlast)


def _imb_body(pb_ref, np_ref, o_ref):
    net = pb_ref[...] + np_ref[0, :, 0:1] + np_ref[1, :, 0:1]
    o_ref[...] = jnp.sum(jnp.abs(net), keepdims=True)


def _imb_call(pb, netp):
    return pl.pallas_call(
        _imb_body,
        out_shape=jax.ShapeDtypeStruct((1, 1), _F32),
    )(pb, netp)


def _zero_rows(buf, nrows):
    z = jnp.zeros((HH,), _F32)

    def body(r, _):
        buf[r] = z
        return 0

    lax.fori_loop(0, nrows, body, 0)


def _make_edge_kernel(with_cnt):
    mesh = plsc.VectorSubcoreMesh(core_axis_name="c", subcore_axis_name="s", num_cores=NC, num_subcores=NS)
    out_type = [jax.ShapeDtypeStruct((2, NN, HH), _F32)]
    scratch = [
        pltpu.VMEM_SHARED((NN, HH), _F32),
        pltpu.VMEM_SHARED((NN, HH), _F32),
        pltpu.VMEM_SHARED((NN, HH), _F32),
        pltpu.VMEM((NPT, HH), _F32),
        pltpu.VMEM((CK,), jnp.int32),
        pltpu.VMEM((CK,), jnp.int32),
        pltpu.VMEM((CK, HH), _F32),
        pltpu.VMEM((CK, HH), _F32),
        pltpu.VMEM((CK, HH), _F32),
        pltpu.VMEM((CK, HH), _F32),
        pltpu.SemaphoreType.DMA,
        pltpu.SemaphoreType.DMA,
        pltpu.SemaphoreType.DMA,
    ]
    if with_cnt:
        out_type.append(jax.ShapeDtypeStruct((2, NN, HH), _F32))
        scratch.append(pltpu.VMEM_SHARED((NN, HH), _F32))
        scratch.append(pltpu.VMEM((CK, HH), _F32))

    def body(ab_hbm, src_hbm, dst_hbm, ep_hbm, *rest):
        if with_cnt:
            (hag_out, cnt_out, at, bt, hg, stage, sidx, didx, epb, ga, gb, hb,
             sem, sem2, sem3, cg, ones) = rest
        else:
            (hag_out, at, bt, hg, stage, sidx, didx, epb, ga, gb, hb,
             sem, sem2, sem3) = rest
        c = lax.axis_index("c")
        s = lax.axis_index("s")
        nb = s * NPT
        pltpu.sync_copy(ab_hbm.at[0, pl.ds(nb, NPT)], stage)
        pltpu.sync_copy(stage, at.at[pl.ds(nb, NPT)])
        pltpu.sync_copy(ab_hbm.at[1, pl.ds(nb, NPT)], stage)
        pltpu.sync_copy(stage, bt.at[pl.ds(nb, NPT)])
        _zero_rows(stage, NPT)
        pltpu.sync_copy(stage, hg.at[pl.ds(nb, NPT)])
        if with_cnt:
            pltpu.sync_copy(stage, cg.at[pl.ds(nb, NPT)])
            one = jnp.ones((HH,), _F32)

            def fill_ones(r, _):
                ones[r] = one
                return 0

            lax.fori_loop(0, CK, fill_ones, 0)
        plsc.subcore_barrier()

        eb0 = (c * NS + s) * EPW

        def chunk(ci, _):
            base = eb0 + ci * CK
            c1 = pltpu.async_copy(src_hbm.at[pl.ds(base, CK)], sidx, sem)
            c2 = pltpu.async_copy(dst_hbm.at[pl.ds(base, CK)], didx, sem2)
            c3 = pltpu.async_copy(ep_hbm.at[pl.ds(base, CK)], epb, sem3)
            c1.wait()
            c2.wait()
            g1 = pltpu.async_copy(at.at[sidx], ga, sem)
            g2 = pltpu.async_copy(bt.at[didx], gb, sem2)
            c3.wait()
            g1.wait()
            g2.wait()

            def edge(k, _):
                h = jnp.maximum(ga[k] + gb[k] + epb[k], 0.0)
                u = lax.bitcast_convert_type(h, jnp.int32)
                u = u + 0x7FFF + ((u >> 16) & 1)
                u = u & jnp.int32(-65536)
                hb[k] = lax.bitcast_convert_type(u, _F32)
                return 0

            lax.fori_loop(0, CK, edge, 0)
            pltpu.sync_copy(hb, hg.at[didx], add=True)
            if with_cnt:
                pltpu.sync_copy(ones, cg.at[didx], add=True)
            return 0

        lax.fori_loop(0, NCHUNK, chunk, 0)
        plsc.subcore_barrier()
        pltpu.sync_copy(hg.at[pl.ds(nb, NPT)], stage)
        pltpu.sync_copy(stage, hag_out.at[c, pl.ds(nb, NPT)])
        if with_cnt:
            pltpu.sync_copy(cg.at[pl.ds(nb, NPT)], stage)
            pltpu.sync_copy(stage, cnt_out.at[c, pl.ds(nb, NPT)])

    return pl.kernel(
        body,
        out_type=tuple(out_type) if with_cnt else out_type[0],
        mesh=mesh,
        scratch_types=scratch,
        compiler_params=pltpu.CompilerParams(use_tc_tiling_on_sc=False, needs_layout_passes=False),
    )


def _make_flow_kernel():
    mesh = plsc.VectorSubcoreMesh(core_axis_name="c", subcore_axis_name="s", num_cores=NC, num_subcores=NS)
    scratch = [
        pltpu.VMEM_SHARED((NN, HH), _F32),
        pltpu.VMEM((NN,), _F32),
        pltpu.VMEM((NPT, HH), _F32),
        pltpu.VMEM((CK,), jnp.int32),
        pltpu.VMEM((CK,), jnp.int32),
        pltpu.VMEM((CK,), _F32),
        pltpu.VMEM((CK,), _F32),
        pltpu.VMEM((CK, HH), _F32),
        pltpu.VMEM((CK, HH), _F32),
    ]

    def body(pb_hbm, src_hbm, dst_hbm, ea0_hbm, flow_out, netp_out,
             neta, pbuf, stage, sidx, didx, ea0b, flb, posr, negr):
        c = lax.axis_index("c")
        s = lax.axis_index("s")
        nb = s * NPT
        pltpu.sync_copy(pb_hbm, pbuf)
        _zero_rows(stage, NPT)
        pltpu.sync_copy(stage, neta.at[pl.ds(nb, NPT)])
        plsc.subcore_barrier()

        eb0 = (c * NS + s) * EPW

        def chunk(ci, _):
            base = eb0 + ci * CK
            pltpu.sync_copy(src_hbm.at[pl.ds(base, CK)], sidx)
            pltpu.sync_copy(dst_hbm.at[pl.ds(base, CK)], didx)
            pltpu.sync_copy(ea0_hbm.at[pl.ds(base, CK)], ea0b)
            for j in range(CK // HH):
                sv = sidx[pl.ds(j * HH, HH)]
                dv = didx[pl.ds(j * HH, HH)]
                ps = plsc.load_gather(pbuf, [sv])
                pd = plsc.load_gather(pbuf, [dv])
                fl = (ps - pd) * ea0b[pl.ds(j * HH, HH)]
                flb[pl.ds(j * HH, HH)] = fl
                for e in range(HH):
                    fv = jnp.broadcast_to(fl[e], (HH,))
                    posr[j * HH + e] = fv
                    negr[j * HH + e] = -fv
            pltpu.sync_copy(flb, flow_out.at[pl.ds(base, CK)])
            pltpu.sync_copy(posr, neta.at[didx], add=True)
            pltpu.sync_copy(negr, neta.at[sidx], add=True)
            return 0

        lax.fori_loop(0, NCHUNK, chunk, 0)
        plsc.subcore_barrier()
        pltpu.sync_copy(neta.at[pl.ds(nb, NPT)], stage)
        pltpu.sync_copy(stage, netp_out.at[c, pl.ds(nb, NPT)])

    return pl.kernel(
        body,
        out_type=(
            jax.ShapeDtypeStruct((EE,), _F32),
            jax.ShapeDtypeStruct((2, NN, HH), _F32),
        ),
        mesh=mesh,
        scratch_types=scratch,
        compiler_params=pltpu.CompilerParams(use_tc_tiling_on_sc=False, needs_layout_passes=False),
    )


_SC_KERNELS = {}


def _get_sc_kernels():
    if not _SC_KERNELS:
        _SC_KERNELS["edge_first"] = _make_edge_kernel(with_cnt=True)
        _SC_KERNELS["edge_rest"] = _make_edge_kernel(with_cnt=False)
        _SC_KERNELS["flow"] = _make_flow_kernel()
    return _SC_KERNELS


def kernel(x, edge_index, edge_attr, W1, b1, W2, b2, Wf, bf):
    src = edge_index[0]
    dst = edge_index[1]
    w1a = W1[:, :LL, :]
    w1b = W1[:, LL:2 * LL, :]
    w1x = W1[:, 2 * LL:2 * LL + DNF, :]
    w1e = W1[:, 2 * LL + DNF:, :]
    xlast = x[:, DNF - 1:DNF]
    ea0 = edge_attr[:, 0]

    sck = _get_sc_kernels()
    xp = _xp_call(x, w1x)
    eap = edge_attr.reshape(_EPK, 8 * DEF)
    wbig = jnp.stack(
        [jax.scipy.linalg.block_diag(*([w1e[i]] * 8)) for i in range(NCV)])
    btile = jnp.tile(b1, (1, 8)).reshape(NCV, 1, 8 * HH)
    ep_all = _ep_call(eap, wbig, btile).reshape(NCV, EE, HH)

    ab = jnp.stack([xp[0], jnp.zeros((NN, HH), _F32)])
    hag, cnt16 = sck["edge_first"](ab, src, dst, ep_all[0])
    for i in range(1, NCV):
        ab = _node_call(hag, cnt16, xp[i], w1a[i], w1b[i], W2[i - 1],
                        b2[i - 1].reshape(1, LL))
        hag = sck["edge_rest"](ab, src, dst, ep_all[i])

    p, pb = _fin_node_call(hag, cnt16, W2[NCV - 1], b2[NCV - 1].reshape(1, LL),
                           Wf, bf.reshape(1, 1), xlast)

    flow, netp = sck["flow"](pb.reshape(NN), src, dst, ea0)
    imb = _imb_call(pb, netp)
    return p, flow.reshape(EE, 1), imb.reshape(1)


# speedup vs baseline: 6.5308x; 1.0337x over previous
import functools

import jax
import jax.numpy as jnp
from jax import lax
from jax.experimental import pallas as pl
from jax.experimental.pallas import tpu as pltpu
from jax.experimental.pallas import tpu_sc as plsc

NN = 10000
EE = 320000
DNF = 128
DEF = 4
LL = 6
HH = 16
NCV = 8

NC = 2
NS = 16
NW = NC * NS
NPT = NN // NS
EPW = EE // NW
CK = 80
NCHUNK = EPW // CK

_F32 = jnp.float32


_BF16 = jnp.bfloat16


def _bf(v):
    return v.astype(_BF16).astype(_F32)


def _xp_body(x_ref, w_ref, o_ref):
    o_ref[0] = jnp.dot(x_ref[...].astype(_BF16), w_ref[0].astype(_BF16),
                       preferred_element_type=_F32)


def _xp_call(x, w1x):
    return pl.pallas_call(
        _xp_body,
        grid=(NCV,),
        in_specs=[
            pl.BlockSpec((NN, DNF), lambda i: (0, 0)),
            pl.BlockSpec((1, DNF, HH), lambda i: (i, 0, 0)),
        ],
        out_specs=pl.BlockSpec((1, NN, HH), lambda i: (i, 0, 0)),
        out_shape=jax.ShapeDtypeStruct((NCV, NN, HH), _F32),
    )(x, w1x)


def _ep_body(ea_ref, w_ref, b_ref, o_ref):
    o_ref[0] = (jnp.dot(ea_ref[...].astype(_BF16), w_ref[0].astype(_BF16),
                        preferred_element_type=_F32)
                + b_ref[0])


_EPK = EE // 8
_EBLKP = 5000


def _ep_call(ea_packed, wbig, btile):
    return pl.pallas_call(
        _ep_body,
        grid=(NCV, _EPK // _EBLKP),
        in_specs=[
            pl.BlockSpec((_EBLKP, 8 * DEF), lambda i, e: (e, 0)),
            pl.BlockSpec((1, 8 * DEF, 8 * HH), lambda i, e: (i, 0, 0)),
            pl.BlockSpec((1, 1, 8 * HH), lambda i, e: (i, 0, 0)),
        ],
        out_specs=pl.BlockSpec((1, _EBLKP, 8 * HH), lambda i, e: (i, e, 0)),
        out_shape=jax.ShapeDtypeStruct((NCV, _EPK, 8 * HH), _F32),
    )(ea_packed, wbig, btile)


def _fma_mm(a, w_ref, k, round_a=True):
    av = _bf(a) if round_a else a
    acc = av[:, 0:1] * _bf(w_ref[0:1, :])
    for j in range(1, k):
        acc = acc + av[:, j:j + 1] * _bf(w_ref[j:j + 1, :])
    return acc


def _node_body(hag_ref, cnt_ref, xp_ref, w1a_ref, w1b_ref, w2_ref, b2_ref,
               ab_ref):
    hs = hag_ref[0] + hag_ref[1]
    cnt = jnp.maximum(cnt_ref[0, :, 0:1] + cnt_ref[1, :, 0:1], 1.0)
    xn = jax.nn.relu(_fma_mm(hs / cnt, w2_ref, HH, round_a=False)
                     + b2_ref[0:1, :])
    ab_ref[0] = _fma_mm(xn, w1a_ref, LL) + xp_ref[...]
    ab_ref[1] = _fma_mm(xn, w1b_ref, LL)


_NBLK = 2000


def _node_call(hag, cnt16, xp, w1a, w1b, w2, b2):
    return pl.pallas_call(
        _node_body,
        grid=(NN // _NBLK,),
        in_specs=[
            pl.BlockSpec((2, _NBLK, HH), lambda g: (0, g, 0)),
            pl.BlockSpec((2, _NBLK, HH), lambda g: (0, g, 0)),
            pl.BlockSpec((_NBLK, HH), lambda g: (g, 0)),
            pl.BlockSpec((LL, HH), lambda g: (0, 0)),
            pl.BlockSpec((LL, HH), lambda g: (0, 0)),
            pl.BlockSpec((HH, LL), lambda g: (0, 0)),
            pl.BlockSpec((1, LL), lambda g: (0, 0)),
        ],
        out_specs=pl.BlockSpec((2, _NBLK, HH), lambda g: (0, g, 0)),
        out_shape=jax.ShapeDtypeStruct((2, NN, HH), _F32),
    )(hag, cnt16, xp, w1a, w1b, w2, b2)


def _fin_node_body(hag_ref, cnt_ref, w2_ref, b2_ref, wf_ref, bf_ref, xl_ref,
                   p_ref, pb_ref):
    hs = hag_ref[0] + hag_ref[1]
    cnt = jnp.maximum(cnt_ref[0, :, 0:1] + cnt_ref[1, :, 0:1], 1.0)
    xn = jax.nn.relu(_fma_mm(hs / cnt, w2_ref, HH, round_a=False)
                     + b2_ref[0:1, :])
    p = jax.nn.relu(_fma_mm(xn, wf_ref, LL) + bf_ref[0:1, :])
    p_ref[...] = p
    xl = xl_ref[...]
    pb_ref[...] = jnp.where(xl != 0.0, xl, p)


def _fin_node_call(hag, cnt16, w2, b2, wf, bf, xlast):
    return pl.pallas_call(
        _fin_node_body,
        grid=(NN // _NBLK,),
        in_specs=[
            pl.BlockSpec((2, _NBLK, HH), lambda g: (0, g, 0)),
            pl.BlockSpec((2, _NBLK, HH), lambda g: (0, g, 0)),
            pl.BlockSpec((HH, LL), lambda g: (0, 0)),
            pl.BlockSpec((1, LL), lambda g: (0, 0)),
            pl.BlockSpec((LL, 1), lambda g: (0, 0)),
            pl.BlockSpec((1, 1), lambda g: (0, 0)),
            pl.BlockSpec((_NBLK, 1), lambda g: (g, 0)),
        ],
        out_specs=(
            pl.BlockSpec((_NBLK, 1), lambda g: (g, 0)),
            pl.BlockSpec((_NBLK, 1), lambda g: (g, 0)),
        ),
        out_shape=(
            jax.ShapeDtypeStruct((NN, 1), _F32),
            jax.ShapeDtypeStruct((NN, 1), _F32),
        ),
    )(hag, cnt16, w2, b2, wf, bf, xlast)


def _imb_body(pb_ref, np_ref, o_ref):
    net = pb_ref[...] + np_ref[0, :, 0:1] + np_ref[1, :, 0:1]
    o_ref[...] = jnp.sum(jnp.abs(net), keepdims=True)


def _imb_call(pb, netp):
    return pl.pallas_call(
        _imb_body,
        out_shape=jax.ShapeDtypeStruct((1, 1), _F32),
    )(pb, netp)


def _zero_rows(buf, nrows):
    z = jnp.zeros((HH,), _F32)

    def body(r, _):
        buf[r] = z
        return 0

    lax.fori_loop(0, nrows, body, 0)


def _make_edge_kernel(with_cnt):
    mesh = plsc.VectorSubcoreMesh(core_axis_name="c", subcore_axis_name="s",
                                  num_cores=NC, num_subcores=NS)
    out_type = [jax.ShapeDtypeStruct((2, NN, HH), _F32)]
    scratch = [
        pltpu.VMEM_SHARED((NN, HH), _F32),
        pltpu.VMEM_SHARED((NN, HH), _F32),
        pltpu.VMEM_SHARED((NN, HH), _F32),
        pltpu.VMEM((NPT, HH), _F32),
    ]
    scratch += [pltpu.VMEM((CK,), jnp.int32)] * 4
    scratch += [pltpu.VMEM((CK,), jnp.int32)] * 2
    scratch += [pltpu.VMEM((CK, HH), _F32)] * 8
    scratch += [pltpu.SemaphoreType.DMA] * 6
    if with_cnt:
        out_type.append(jax.ShapeDtypeStruct((2, NN, HH), _F32))
        scratch.append(pltpu.VMEM_SHARED((NN, HH), _F32))
        scratch.append(pltpu.VMEM((CK, HH), _F32))
        scratch += [pltpu.SemaphoreType.DMA] * 2

    def body(ab_hbm, src_hbm, dst_hbm, ep_hbm, *rest):
        if with_cnt:
            (hag_out, cnt_out, at, bt, hg, stage,
             sidx0, sidx1, didx0, didx1, dsc0, dsc1,
             epb0, epb1, ga0, ga1, gb0, gb1, hb0, hb1,
             sml0, sml1, smg0, smg1, sms0, sms1,
             cg, ones, smc0, smc1) = rest
            smc = [smc0, smc1]
        else:
            (hag_out, at, bt, hg, stage,
             sidx0, sidx1, didx0, didx1, dsc0, dsc1,
             epb0, epb1, ga0, ga1, gb0, gb1, hb0, hb1,
             sml0, sml1, smg0, smg1, sms0, sms1) = rest
        sidx = [sidx0, sidx1]
        didx = [didx0, didx1]
        dsc = [dsc0, dsc1]
        epb = [epb0, epb1]
        ga = [ga0, ga1]
        gb = [gb0, gb1]
        hb = [hb0, hb1]
        sml = [sml0, sml1]
        smg = [smg0, smg1]
        sms = [sms0, sms1]
        cc = lax.axis_index("c")
        ss = lax.axis_index("s")
        nb = ss * NPT
        eb0 = (cc * NS + ss) * EPW

        def load_idx(ci, b):
            base = eb0 + ci * CK
            pltpu.make_async_copy(src_hbm.at[pl.ds(base, CK)], sidx[b],
                                  sml[b]).start()
            pltpu.make_async_copy(dst_hbm.at[pl.ds(base, CK)], didx[b],
                                  sml[b]).start()

        def load_ep(ci, b):
            base = eb0 + ci * CK
            pltpu.make_async_copy(ep_hbm.at[pl.ds(base, CK)], epb[b],
                                  sml[b]).start()

        def wait_load3(b):
            pltpu.make_async_copy(src_hbm.at[pl.ds(0, CK)], sidx[b],
                                  sml[b]).wait()
            pltpu.make_async_copy(src_hbm.at[pl.ds(0, CK)], didx[b],
                                  sml[b]).wait()
            pltpu.make_async_copy(ep_hbm.at[pl.ds(0, CK)], epb[b],
                                  sml[b]).wait()

        def gathers(b):
            pltpu.make_async_copy(at.at[sidx[b]], ga[b], smg[b]).start()
            pltpu.make_async_copy(bt.at[didx[b]], gb[b], smg[b]).start()

        def wait_gathers(b):
            pltpu.make_async_copy(ep_hbm.at[pl.ds(0, CK)], ga[b],
                                  smg[b]).wait()
            pltpu.make_async_copy(ep_hbm.at[pl.ds(0, CK)], gb[b],
                                  smg[b]).wait()

        def scatter(b):
            pltpu.make_async_copy(hb[b], hg.at[dsc[b]], sms[b]).start(add=True)
            if with_cnt:
                pltpu.make_async_copy(ones, cg.at[dsc[b]],
                                      smc[b]).start(add=True)

        def wait_scatter(b):
            pltpu.make_async_copy(ep_hbm.at[pl.ds(0, CK)], hb[b],
                                  sms[b]).wait()
            if with_cnt:
                pltpu.make_async_copy(ep_hbm.at[pl.ds(0, CK)], ones,
                                      smc[b]).wait()

        def copy_didx_to_dsc(b):
            for q in range(CK // HH):
                dsc[b][pl.ds(q * HH, HH)] = didx[b][pl.ds(q * HH, HH)]

        def compute(b):
            def edge(k, _):
                h = jnp.maximum(ga[b][k] + gb[b][k] + epb[b][k], 0.0)
                u = lax.bitcast_convert_type(h, jnp.int32)
                u = u + 0x7FFF + ((u >> 16) & 1)
                u = u & jnp.int32(-65536)
                hb[b][k] = lax.bitcast_convert_type(u, _F32)
                return 0

            lax.fori_loop(0, CK, edge, 0)

        load_idx(0, 0)
        load_ep(0, 0)
        load_idx(1, 1)
        load_ep(1, 1)

        pltpu.sync_copy(ab_hbm.at[0, pl.ds(nb, NPT)], stage)
        pltpu.sync_copy(stage, at.at[pl.ds(nb, NPT)])
        pltpu.sync_copy(ab_hbm.at[1, pl.ds(nb, NPT)], stage)
        pltpu.sync_copy(stage, bt.at[pl.ds(nb, NPT)])
        _zero_rows(stage, NPT)
        pltpu.sync_copy(stage, hg.at[pl.ds(nb, NPT)])
        if with_cnt:
            pltpu.sync_copy(stage, cg.at[pl.ds(nb, NPT)])
            one = jnp.ones((HH,), _F32)

            def fill_ones(r, _):
                ones[r] = one
                return 0

            lax.fori_loop(0, CK, fill_ones, 0)
        plsc.subcore_barrier()

        wait_load3(0)
        gathers(0)

        def half(t, ci, b):
            wait_load3(1 - b)
            gathers(1 - b)
            wait_gathers(b)

            @pl.when(t > 0)
            def _():
                wait_scatter(b)

            copy_didx_to_dsc(b)

            @pl.when(ci + 2 < NCHUNK)
            def _():
                load_idx(ci + 2, b)

            compute(b)
            scatter(b)

            @pl.when(ci + 2 < NCHUNK)
            def _():
                load_ep(ci + 2, b)

        def pair(t, _):
            half(t, 2 * t, 0)
            half(t, 2 * t + 1, 1)
            return 0

        lax.fori_loop(0, NCHUNK // 2, pair, 0)

        tci = NCHUNK - 1
        wait_gathers(0)
        wait_scatter(0)
        copy_didx_to_dsc(0)
        compute(0)
        scatter(0)
        wait_scatter(1)
        wait_scatter(0)

        plsc.subcore_barrier()
        pltpu.sync_copy(hg.at[pl.ds(nb, NPT)], stage)
        pltpu.sync_copy(stage, hag_out.at[cc, pl.ds(nb, NPT)])
        if with_cnt:
            pltpu.sync_copy(cg.at[pl.ds(nb, NPT)], stage)
            pltpu.sync_copy(stage, cnt_out.at[cc, pl.ds(nb, NPT)])

    return pl.kernel(
        body,
        out_type=tuple(out_type) if with_cnt else out_type[0],
        mesh=mesh,
        scratch_types=scratch,
        compiler_params=pltpu.CompilerParams(use_tc_tiling_on_sc=False,
                                             needs_layout_passes=False),
    )


def _make_flow_kernel():
    mesh = plsc.VectorSubcoreMesh(core_axis_name="c", subcore_axis_name="s", num_cores=NC, num_subcores=NS)
    scratch = [
        pltpu.VMEM_SHARED((NN, HH), _F32),
        pltpu.VMEM((NN,), _F32),
        pltpu.VMEM((NPT, HH), _F32),
        pltpu.VMEM((CK,), jnp.int32),
        pltpu.VMEM((CK,), jnp.int32),
        pltpu.VMEM((CK,), _F32),
        pltpu.VMEM((CK,), _F32),
        pltpu.VMEM((CK, HH), _F32),
        pltpu.VMEM((CK, HH), _F32),
    ]

    def body(pb_hbm, src_hbm, dst_hbm, ea0_hbm, flow_out, netp_out,
             neta, pbuf, stage, sidx, didx, ea0b, flb, posr, negr):
        c = lax.axis_index("c")
        s = lax.axis_index("s")
        nb = s * NPT
        pltpu.sync_copy(pb_hbm, pbuf)
        _zero_rows(stage, NPT)
        pltpu.sync_copy(stage, neta.at[pl.ds(nb, NPT)])
        plsc.subcore_barrier()

        eb0 = (c * NS + s) * EPW

        def chunk(ci, _):
            base = eb0 + ci * CK
            pltpu.sync_copy(src_hbm.at[pl.ds(base, CK)], sidx)
            pltpu.sync_copy(dst_hbm.at[pl.ds(base, CK)], didx)
            pltpu.sync_copy(ea0_hbm.at[pl.ds(base, CK)], ea0b)
            for j in range(CK // HH):
                sv = sidx[pl.ds(j * HH, HH)]
                dv = didx[pl.ds(j * HH, HH)]
                ps = plsc.load_gather(pbuf, [sv])
                pd = plsc.load_gather(pbuf, [dv])
                fl = (ps - pd) * ea0b[pl.ds(j * HH, HH)]
                flb[pl.ds(j * HH, HH)] = fl
                for e in range(HH):
                    fv = jnp.broadcast_to(fl[e], (HH,))
                    posr[j * HH + e] = fv
                    negr[j * HH + e] = -fv
            pltpu.sync_copy(flb, flow_out.at[pl.ds(base, CK)])
            pltpu.sync_copy(posr, neta.at[didx], add=True)
            pltpu.sync_copy(negr, neta.at[sidx], add=True)
            return 0

        lax.fori_loop(0, NCHUNK, chunk, 0)
        plsc.subcore_barrier()
        pltpu.sync_copy(neta.at[pl.ds(nb, NPT)], stage)
        pltpu.sync_copy(stage, netp_out.at[c, pl.ds(nb, NPT)])

    return pl.kernel(
        body,
        out_type=(
            jax.ShapeDtypeStruct((EE,), _F32),
            jax.ShapeDtypeStruct((2, NN, HH), _F32),
        ),
        mesh=mesh,
        scratch_types=scratch,
        compiler_params=pltpu.CompilerParams(use_tc_tiling_on_sc=False, needs_layout_passes=False),
    )


_SC_KERNELS = {}


def _get_sc_kernels():
    if not _SC_KERNELS:
        _SC_KERNELS["edge_first"] = _make_edge_kernel(with_cnt=True)
        _SC_KERNELS["edge_rest"] = _make_edge_kernel(with_cnt=False)
        _SC_KERNELS["flow"] = _make_flow_kernel()
    return _SC_KERNELS


def kernel(x, edge_index, edge_attr, W1, b1, W2, b2, Wf, bf):
    src = edge_index[0]
    dst = edge_index[1]
    w1a = W1[:, :LL, :]
    w1b = W1[:, LL:2 * LL, :]
    w1x = W1[:, 2 * LL:2 * LL + DNF, :]
    w1e = W1[:, 2 * LL + DNF:, :]
    xlast = x[:, DNF - 1:DNF]
    ea0 = edge_attr[:, 0]

    sck = _get_sc_kernels()
    xp = _xp_call(x, w1x)
    eap = edge_attr.reshape(_EPK, 8 * DEF)
    wbig = jnp.stack(
        [jax.scipy.linalg.block_diag(*([w1e[i]] * 8)) for i in range(NCV)])
    btile = jnp.tile(b1, (1, 8)).reshape(NCV, 1, 8 * HH)
    ep_all = _ep_call(eap, wbig, btile).reshape(NCV, EE, HH)

    ab = jnp.stack([xp[0], jnp.zeros((NN, HH), _F32)])
    hag, cnt16 = sck["edge_first"](ab, src, dst, ep_all[0])
    for i in range(1, NCV):
        ab = _node_call(hag, cnt16, xp[i], w1a[i], w1b[i], W2[i - 1],
                        b2[i - 1].reshape(1, LL))
        hag = sck["edge_rest"](ab, src, dst, ep_all[i])

    p, pb = _fin_node_call(hag, cnt16, W2[NCV - 1], b2[NCV - 1].reshape(1, LL),
                           Wf, bf.reshape(1, 1), xlast)

    flow, netp = sck["flow"](pb.reshape(NN), src, dst, ea0)
    imb = _imb_call(pb, netp)
    return p, flow.reshape(EE, 1), imb.reshape(1)


# speedup vs baseline: 6.5308x; 1.0000x over previous
import functools

import jax
import jax.numpy as jnp
from jax import lax
from jax.experimental import pallas as pl
from jax.experimental.pallas import tpu as pltpu
from jax.experimental.pallas import tpu_sc as plsc

NN = 10000
EE = 320000
DNF = 128
DEF = 4
LL = 6
HH = 16
NCV = 8

NC = 2
NS = 16
NW = NC * NS
NPT = NN // NS
EPW = EE // NW
CK = 80
NCHUNK = EPW // CK

_F32 = jnp.float32


_BF16 = jnp.bfloat16


def _bf(v):
    return v.astype(_BF16).astype(_F32)


def _xp_body(x_ref, w_ref, o_ref):
    o_ref[0] = jnp.dot(x_ref[...].astype(_BF16), w_ref[0].astype(_BF16),
                       preferred_element_type=_F32)


def _xp_call(x, w1x):
    return pl.pallas_call(
        _xp_body,
        grid=(NCV,),
        in_specs=[
            pl.BlockSpec((NN, DNF), lambda i: (0, 0)),
            pl.BlockSpec((1, DNF, HH), lambda i: (i, 0, 0)),
        ],
        out_specs=pl.BlockSpec((1, NN, HH), lambda i: (i, 0, 0)),
        out_shape=jax.ShapeDtypeStruct((NCV, NN, HH), _F32),
    )(x, w1x)


def _ep_body(ea_ref, w_ref, b_ref, o_ref):
    o_ref[0] = (jnp.dot(ea_ref[...].astype(_BF16), w_ref[0].astype(_BF16),
                        preferred_element_type=_F32)
                + b_ref[0])


_EPK = EE // 8
_EBLKP = 5000


def _ep_call(ea_packed, wbig, btile):
    return pl.pallas_call(
        _ep_body,
        grid=(NCV, _EPK // _EBLKP),
        in_specs=[
            pl.BlockSpec((_EBLKP, 8 * DEF), lambda i, e: (e, 0)),
            pl.BlockSpec((1, 8 * DEF, 8 * HH), lambda i, e: (i, 0, 0)),
            pl.BlockSpec((1, 1, 8 * HH), lambda i, e: (i, 0, 0)),
        ],
        out_specs=pl.BlockSpec((1, _EBLKP, 8 * HH), lambda i, e: (i, e, 0)),
        out_shape=jax.ShapeDtypeStruct((NCV, _EPK, 8 * HH), _F32),
    )(ea_packed, wbig, btile)


def _fma_mm(a, w_ref, k, round_a=True):
    av = _bf(a) if round_a else a
    acc = av[:, 0:1] * _bf(w_ref[0:1, :])
    for j in range(1, k):
        acc = acc + av[:, j:j + 1] * _bf(w_ref[j:j + 1, :])
    return acc


def _node_body(hag_ref, cnt_ref, xp_ref, w1a_ref, w1b_ref, w2_ref, b2_ref,
               ab_ref):
    hs = hag_ref[0] + hag_ref[1]
    cnt = jnp.maximum(cnt_ref[0, :, 0:1] + cnt_ref[1, :, 0:1], 1.0)
    xn = jax.nn.relu(_fma_mm(hs / cnt, w2_ref, HH, round_a=False)
                     + b2_ref[0:1, :])
    ab_ref[0] = _fma_mm(xn, w1a_ref, LL) + xp_ref[...]
    ab_ref[1] = _fma_mm(xn, w1b_ref, LL)


_NBLK = 2000


def _node_call(hag, cnt16, xp, w1a, w1b, w2, b2):
    return pl.pallas_call(
        _node_body,
        grid=(NN // _NBLK,),
        in_specs=[
            pl.BlockSpec((2, _NBLK, HH), lambda g: (0, g, 0)),
            pl.BlockSpec((2, _NBLK, HH), lambda g: (0, g, 0)),
            pl.BlockSpec((_NBLK, HH), lambda g: (g, 0)),
            pl.BlockSpec((LL, HH), lambda g: (0, 0)),
            pl.BlockSpec((LL, HH), lambda g: (0, 0)),
            pl.BlockSpec((HH, LL), lambda g: (0, 0)),
            pl.BlockSpec((1, LL), lambda g: (0, 0)),
        ],
        out_specs=pl.BlockSpec((2, _NBLK, HH), lambda g: (0, g, 0)),
        out_shape=jax.ShapeDtypeStruct((2, NN, HH), _F32),
    )(hag, cnt16, xp, w1a, w1b, w2, b2)


def _fin_node_body(hag_ref, cnt_ref, w2_ref, b2_ref, wf_ref, bf_ref, xl_ref,
                   p_ref, pb_ref):
    hs = hag_ref[0] + hag_ref[1]
    cnt = jnp.maximum(cnt_ref[0, :, 0:1] + cnt_ref[1, :, 0:1], 1.0)
    xn = jax.nn.relu(_fma_mm(hs / cnt, w2_ref, HH, round_a=False)
                     + b2_ref[0:1, :])
    p = jax.nn.relu(_fma_mm(xn, wf_ref, LL) + bf_ref[0:1, :])
    p_ref[...] = p
    xl = xl_ref[...]
    pb_ref[...] = jnp.where(xl != 0.0, xl, p)


def _fin_node_call(hag, cnt16, w2, b2, wf, bf, xlast):
    return pl.pallas_call(
        _fin_node_body,
        grid=(NN // _NBLK,),
        in_specs=[
            pl.BlockSpec((2, _NBLK, HH), lambda g: (0, g, 0)),
            pl.BlockSpec((2, _NBLK, HH), lambda g: (0, g, 0)),
            pl.BlockSpec((HH, LL), lambda g: (0, 0)),
            pl.BlockSpec((1, LL), lambda g: (0, 0)),
            pl.BlockSpec((LL, 1), lambda g: (0, 0)),
            pl.BlockSpec((1, 1), lambda g: (0, 0)),
            pl.BlockSpec((_NBLK, 1), lambda g: (g, 0)),
        ],
        out_specs=(
            pl.BlockSpec((_NBLK, 1), lambda g: (g, 0)),
            pl.BlockSpec((_NBLK, 1), lambda g: (g, 0)),
        ),
        out_shape=(
            jax.ShapeDtypeStruct((NN, 1), _F32),
            jax.ShapeDtypeStruct((NN, 1), _F32),
        ),
    )(hag, cnt16, w2, b2, wf, bf, xlast)


def _imb_body(pb_ref, np_ref, o_ref):
    net = pb_ref[...] + np_ref[0, :, 0:1] + np_ref[1, :, 0:1]
    o_ref[...] = jnp.sum(jnp.abs(net), keepdims=True)


def _imb_call(pb, netp):
    return pl.pallas_call(
        _imb_body,
        out_shape=jax.ShapeDtypeStruct((1, 1), _F32),
    )(pb, netp)


def _zero_rows(buf, nrows):
    z = jnp.zeros((HH,), _F32)

    def body(r, _):
        buf[r] = z
        return 0

    lax.fori_loop(0, nrows, body, 0)


def _make_edge_kernel(with_cnt):
    mesh = plsc.VectorSubcoreMesh(core_axis_name="c", subcore_axis_name="s",
                                  num_cores=NC, num_subcores=NS)
    out_type = [jax.ShapeDtypeStruct((2, NN, HH), _F32)]
    scratch = [
        pltpu.VMEM_SHARED((NN, HH), _F32),
        pltpu.VMEM_SHARED((NN, HH), _F32),
        pltpu.VMEM_SHARED((NN, HH), _F32),
        pltpu.VMEM((NPT, HH), _F32),
    ]
    scratch += [pltpu.VMEM((CK,), jnp.int32)] * 4
    scratch += [pltpu.VMEM((CK,), jnp.int32)] * 2
    scratch += [pltpu.VMEM((CK, HH), _F32)] * 8
    scratch += [pltpu.SemaphoreType.DMA] * 6
    if with_cnt:
        out_type.append(jax.ShapeDtypeStruct((2, NN, HH), _F32))
        scratch.append(pltpu.VMEM_SHARED((NN, HH), _F32))
        scratch.append(pltpu.VMEM((CK, HH), _F32))
        scratch += [pltpu.SemaphoreType.DMA] * 2

    def body(ab_hbm, src_hbm, dst_hbm, ep_hbm, *rest):
        if with_cnt:
            (hag_out, cnt_out, at, bt, hg, stage,
             sidx0, sidx1, didx0, didx1, dsc0, dsc1,
             epb0, epb1, ga0, ga1, gb0, gb1, hb0, hb1,
             sml0, sml1, smg0, smg1, sms0, sms1,
             cg, ones, smc0, smc1) = rest
            smc = [smc0, smc1]
        else:
            (hag_out, at, bt, hg, stage,
             sidx0, sidx1, didx0, didx1, dsc0, dsc1,
             epb0, epb1, ga0, ga1, gb0, gb1, hb0, hb1,
             sml0, sml1, smg0, smg1, sms0, sms1) = rest
        sidx = [sidx0, sidx1]
        didx = [didx0, didx1]
        dsc = [dsc0, dsc1]
        epb = [epb0, epb1]
        ga = [ga0, ga1]
        gb = [gb0, gb1]
        hb = [hb0, hb1]
        sml = [sml0, sml1]
        smg = [smg0, smg1]
        sms = [sms0, sms1]
        cc = lax.axis_index("c")
        ss = lax.axis_index("s")
        nb = ss * NPT
        eb0 = (cc * NS + ss) * EPW

        def load_idx(ci, b):
            base = eb0 + ci * CK
            pltpu.make_async_copy(src_hbm.at[pl.ds(base, CK)], sidx[b],
                                  sml[b]).start()
            pltpu.make_async_copy(dst_hbm.at[pl.ds(base, CK)], didx[b],
                                  sml[b]).start()

        def load_ep(ci, b):
            base = eb0 + ci * CK
            pltpu.make_async_copy(ep_hbm.at[pl.ds(base, CK)], epb[b],
                                  sml[b]).start()

        def wait_load3(b):
            pltpu.make_async_copy(src_hbm.at[pl.ds(0, CK)], sidx[b],
                                  sml[b]).wait()
            pltpu.make_async_copy(src_hbm.at[pl.ds(0, CK)], didx[b],
                                  sml[b]).wait()
            pltpu.make_async_copy(ep_hbm.at[pl.ds(0, CK)], epb[b],
                                  sml[b]).wait()

        def gathers(b):
            pltpu.make_async_copy(at.at[sidx[b]], ga[b], smg[b]).start()
            pltpu.make_async_copy(bt.at[didx[b]], gb[b], smg[b]).start()

        def wait_gathers(b):
            pltpu.make_async_copy(ep_hbm.at[pl.ds(0, CK)], ga[b],
                                  smg[b]).wait()
            pltpu.make_async_copy(ep_hbm.at[pl.ds(0, CK)], gb[b],
                                  smg[b]).wait()

        def scatter(b):
            pltpu.make_async_copy(hb[b], hg.at[dsc[b]], sms[b]).start(add=True)
            if with_cnt:
                pltpu.make_async_copy(ones, cg.at[dsc[b]],
                                      smc[b]).start(add=True)

        def wait_scatter(b):
            pltpu.make_async_copy(ep_hbm.at[pl.ds(0, CK)], hb[b],
                                  sms[b]).wait()
            if with_cnt:
                pltpu.make_async_copy(ep_hbm.at[pl.ds(0, CK)], ones,
                                      smc[b]).wait()

        def copy_didx_to_dsc(b):
            for q in range(CK // HH):
                dsc[b][pl.ds(q * HH, HH)] = didx[b][pl.ds(q * HH, HH)]

        def compute(b):
            def edge(k, _):
                h = jnp.maximum(ga[b][k] + gb[b][k] + epb[b][k], 0.0)
                u = lax.bitcast_convert_type(h, jnp.int32)
                u = u + 0x7FFF + ((u >> 16) & 1)
                u = u & jnp.int32(-65536)
                hb[b][k] = lax.bitcast_convert_type(u, _F32)
                return 0

            lax.fori_loop(0, CK, edge, 0)

        load_idx(0, 0)
        load_ep(0, 0)
        load_idx(1, 1)
        load_ep(1, 1)

        pltpu.sync_copy(ab_hbm.at[0, pl.ds(nb, NPT)], stage)
        pltpu.sync_copy(stage, at.at[pl.ds(nb, NPT)])
        pltpu.sync_copy(ab_hbm.at[1, pl.ds(nb, NPT)], stage)
        pltpu.sync_copy(stage, bt.at[pl.ds(nb, NPT)])
        _zero_rows(stage, NPT)
        pltpu.sync_copy(stage, hg.at[pl.ds(nb, NPT)])
        if with_cnt:
            pltpu.sync_copy(stage, cg.at[pl.ds(nb, NPT)])
            one = jnp.ones((HH,), _F32)

            def fill_ones(r, _):
                ones[r] = one
                return 0

            lax.fori_loop(0, CK, fill_ones, 0)
        plsc.subcore_barrier()

        wait_load3(0)
        gathers(0)

        def half(t, ci, b):
            wait_load3(1 - b)
            gathers(1 - b)
            wait_gathers(b)

            @pl.when(t > 0)
            def _():
                wait_scatter(b)

            copy_didx_to_dsc(b)

            @pl.when(ci + 2 < NCHUNK)
            def _():
                load_idx(ci + 2, b)

            compute(b)
            scatter(b)

            @pl.when(ci + 2 < NCHUNK)
            def _():
                load_ep(ci + 2, b)

        def pair(t, _):
            half(t, 2 * t, 0)
            half(t, 2 * t + 1, 1)
            return 0

        lax.fori_loop(0, NCHUNK // 2, pair, 0)

        tci = NCHUNK - 1
        wait_gathers(0)
        wait_scatter(0)
        copy_didx_to_dsc(0)
        compute(0)
        scatter(0)
        wait_scatter(1)
        wait_scatter(0)

        plsc.subcore_barrier()
        pltpu.sync_copy(hg.at[pl.ds(nb, NPT)], stage)
        pltpu.sync_copy(stage, hag_out.at[cc, pl.ds(nb, NPT)])
        if with_cnt:
            pltpu.sync_copy(cg.at[pl.ds(nb, NPT)], stage)
            pltpu.sync_copy(stage, cnt_out.at[cc, pl.ds(nb, NPT)])

    return pl.kernel(
        body,
        out_type=tuple(out_type) if with_cnt else out_type[0],
        mesh=mesh,
        scratch_types=scratch,
        compiler_params=pltpu.CompilerParams(use_tc_tiling_on_sc=False,
                                             needs_layout_passes=False,
                                             disable_bounds_checks=True,
                                             skip_device_barrier=True),
    )


def _make_flow_kernel():
    mesh = plsc.VectorSubcoreMesh(core_axis_name="c", subcore_axis_name="s", num_cores=NC, num_subcores=NS)
    scratch = [
        pltpu.VMEM_SHARED((NN, HH), _F32),
        pltpu.VMEM((NN,), _F32),
        pltpu.VMEM((NPT, HH), _F32),
        pltpu.VMEM((CK,), jnp.int32),
        pltpu.VMEM((CK,), jnp.int32),
        pltpu.VMEM((CK,), _F32),
        pltpu.VMEM((CK,), _F32),
        pltpu.VMEM((CK, HH), _F32),
        pltpu.VMEM((CK, HH), _F32),
    ]

    def body(pb_hbm, src_hbm, dst_hbm, ea0_hbm, flow_out, netp_out,
             neta, pbuf, stage, sidx, didx, ea0b, flb, posr, negr):
        c = lax.axis_index("c")
        s = lax.axis_index("s")
        nb = s * NPT
        pltpu.sync_copy(pb_hbm, pbuf)
        _zero_rows(stage, NPT)
        pltpu.sync_copy(stage, neta.at[pl.ds(nb, NPT)])
        plsc.subcore_barrier()

        eb0 = (c * NS + s) * EPW

        def chunk(ci, _):
            base = eb0 + ci * CK
            pltpu.sync_copy(src_hbm.at[pl.ds(base, CK)], sidx)
            pltpu.sync_copy(dst_hbm.at[pl.ds(base, CK)], didx)
            pltpu.sync_copy(ea0_hbm.at[pl.ds(base, CK)], ea0b)
            for j in range(CK // HH):
                sv = sidx[pl.ds(j * HH, HH)]
                dv = didx[pl.ds(j * HH, HH)]
                ps = plsc.load_gather(pbuf, [sv])
                pd = plsc.load_gather(pbuf, [dv])
                fl = (ps - pd) * ea0b[pl.ds(j * HH, HH)]
                flb[pl.ds(j * HH, HH)] = fl
                for e in range(HH):
                    fv = jnp.broadcast_to(fl[e], (HH,))
                    posr[j * HH + e] = fv
                    negr[j * HH + e] = -fv
            pltpu.sync_copy(flb, flow_out.at[pl.ds(base, CK)])
            pltpu.sync_copy(posr, neta.at[didx], add=True)
            pltpu.sync_copy(negr, neta.at[sidx], add=True)
            return 0

        lax.fori_loop(0, NCHUNK, chunk, 0)
        plsc.subcore_barrier()
        pltpu.sync_copy(neta.at[pl.ds(nb, NPT)], stage)
        pltpu.sync_copy(stage, netp_out.at[c, pl.ds(nb, NPT)])

    return pl.kernel(
        body,
        out_type=(
            jax.ShapeDtypeStruct((EE,), _F32),
            jax.ShapeDtypeStruct((2, NN, HH), _F32),
        ),
        mesh=mesh,
        scratch_types=scratch,
        compiler_params=pltpu.CompilerParams(use_tc_tiling_on_sc=False, needs_layout_passes=False, disable_bounds_checks=True, skip_device_barrier=True),
    )


_SC_KERNELS = {}


def _get_sc_kernels():
    if not _SC_KERNELS:
        _SC_KERNELS["edge_first"] = _make_edge_kernel(with_cnt=True)
        _SC_KERNELS["edge_rest"] = _make_edge_kernel(with_cnt=False)
        _SC_KERNELS["flow"] = _make_flow_kernel()
    return _SC_KERNELS


def kernel(x, edge_index, edge_attr, W1, b1, W2, b2, Wf, bf):
    src = edge_index[0]
    dst = edge_index[1]
    w1a = W1[:, :LL, :]
    w1b = W1[:, LL:2 * LL, :]
    w1x = W1[:, 2 * LL:2 * LL + DNF, :]
    w1e = W1[:, 2 * LL + DNF:, :]
    xlast = x[:, DNF - 1:DNF]
    ea0 = edge_attr[:, 0]

    sck = _get_sc_kernels()
    xp = _xp_call(x, w1x)
    eap = edge_attr.reshape(_EPK, 8 * DEF)
    wbig = jnp.stack(
        [jax.scipy.linalg.block_diag(*([w1e[i]] * 8)) for i in range(NCV)])
    btile = jnp.tile(b1, (1, 8)).reshape(NCV, 1, 8 * HH)
    ep_all = _ep_call(eap, wbig, btile).reshape(NCV, EE, HH)

    ab = jnp.stack([xp[0], jnp.zeros((NN, HH), _F32)])
    hag, cnt16 = sck["edge_first"](ab, src, dst, ep_all[0])
    for i in range(1, NCV):
        ab = _node_call(hag, cnt16, xp[i], w1a[i], w1b[i], W2[i - 1],
                        b2[i - 1].reshape(1, LL))
        hag = sck["edge_rest"](ab, src, dst, ep_all[i])

    p, pb = _fin_node_call(hag, cnt16, W2[NCV - 1], b2[NCV - 1].reshape(1, LL),
                           Wf, bf.reshape(1, 1), xlast)

    flow, netp = sck["flow"](pb.reshape(NN), src, dst, ea0)
    imb = _imb_call(pb, netp)
    return p, flow.reshape(EE, 1), imb.reshape(1)
